# baseline (speedup 1.0000x reference)
import functools

import jax
import jax.numpy as jnp
from jax import lax
from jax.experimental import pallas as pl
from jax.experimental.pallas import tpu as pltpu
from jax.experimental.pallas import tpu_sc as plsc

L = 16


@functools.lru_cache(maxsize=None)
def _make_sc_lookup(N, S, D, CH):
    info = plsc.get_sparse_core_info()
    NC, NS = info.num_cores, info.num_subcores
    NW = NC * NS
    TPW = N // NW
    NCH = TPW // CH
    mesh = plsc.VectorSubcoreMesh(core_axis_name="c", subcore_axis_name="s")

    @functools.partial(
        pl.kernel,
        mesh=mesh,
        out_type=jax.ShapeDtypeStruct((N, D), jnp.float32),
        scratch_types=[
            pltpu.VMEM((TPW,), jnp.int32),
            pltpu.VMEM((CH, D), jnp.float32),
            pltpu.VMEM((CH, D), jnp.float32),
            pltpu.SemaphoreType.DMA,
        ],
    )
    def lookup(tokens_hbm, table_hbm, pos_hbm, out_hbm, idx_v, rows_v, pos_v, sem):
        wid = lax.axis_index("s") * NC + lax.axis_index("c")
        base = wid * TPW
        s_base = base % S
        pltpu.sync_copy(tokens_hbm.at[pl.ds(base, TPW)], idx_v)

        def chunk_body(c, carry):
            off = c * CH
            gather = pltpu.async_copy(
                table_hbm.at[idx_v.at[pl.ds(off, CH)]], rows_v, sem
            )
            pltpu.sync_copy(pos_hbm.at[pl.ds(s_base + off, CH)], pos_v)
            gather.wait()

            def row_body(i, inner):
                for j in range(D // L):
                    sl = pl.ds(j * L, L)
                    rows_v[i, sl] = rows_v[i, sl] + pos_v[i, sl]
                return inner

            lax.fori_loop(0, CH, row_body, 0)
            pltpu.sync_copy(rows_v, out_hbm.at[pl.ds(base + off, CH)])
            return carry

        lax.fori_loop(0, NCH, chunk_body, 0)

    return lookup


def kernel(tokens, token_emb, pos_enc):
    B, S = tokens.shape
    D = token_emb.shape[1]
    tok_flat = tokens.reshape(-1).astype(jnp.int32)
    pos2d = pos_enc[0, :S, :]
    out = _make_sc_lookup(B * S, S, D, 32)(tok_flat, token_emb, pos2d)
    return out.reshape(B, S, D)


# speedup vs baseline: 1.2089x; 1.2089x over previous
import functools

import jax
import jax.numpy as jnp
from jax import lax
from jax.experimental import pallas as pl
from jax.experimental.pallas import tpu as pltpu
from jax.experimental.pallas import tpu_sc as plsc

L = 16
NBUF = 3
CH = 16


@functools.lru_cache(maxsize=None)
def _make_sc_lookup(B, S, D):
    info = plsc.get_sparse_core_info()
    NC, NS = info.num_cores, info.num_subcores
    NW = NC * NS
    N = B * S
    SPW = S // NW
    CPB = SPW // CH
    NCHT = B * CPB
    mesh = plsc.VectorSubcoreMesh(core_axis_name="c", subcore_axis_name="s")

    @functools.partial(
        pl.kernel,
        mesh=mesh,
        out_type=jax.ShapeDtypeStruct((N, D), jnp.float32),
        scratch_types=[
            pltpu.VMEM((B * SPW,), jnp.int32),
            pltpu.VMEM((SPW, D), jnp.float32),
            *[pltpu.VMEM((CH, D), jnp.float32) for _ in range(NBUF)],
            pltpu.SemaphoreType.DMA,
            *[pltpu.SemaphoreType.DMA for _ in range(NBUF)],
            *[pltpu.SemaphoreType.DMA for _ in range(NBUF)],
        ],
    )
    def lookup(tokens_hbm, table_hbm, pos_hbm, out_hbm,
               idx_v, pos_v, *bufs_and_sems):
        bufs = bufs_and_sems[:NBUF]
        pos_sem = bufs_and_sems[NBUF]
        gsems = bufs_and_sems[NBUF + 1:2 * NBUF + 1]
        ssems = bufs_and_sems[2 * NBUF + 1:]

        wid = lax.axis_index("s") * NC + lax.axis_index("c")
        s0 = wid * SPW

        pos_cp = pltpu.async_copy(pos_hbm.at[pl.ds(s0, SPW)], pos_v, pos_sem)
        for b in range(B):
            pltpu.sync_copy(tokens_hbm.at[pl.ds(b * S + s0, SPW)],
                            idx_v.at[pl.ds(b * SPW, SPW)])

        def start_gather(t):
            k = t % NBUF
            return pltpu.async_copy(
                table_hbm.at[idx_v.at[pl.ds(t * CH, CH)]], bufs[k], gsems[k])

        gathers = {t: start_gather(t) for t in range(min(2, NCHT))}
        stores = {}
        pos_cp.wait()

        for t in range(NCHT):
            k = t % NBUF
            if t + 2 < NCHT:
                kn = (t + 2) % NBUF
                if t + 2 >= NBUF:
                    stores[t + 2 - NBUF].wait()
                gathers[t + 2] = start_gather(t + 2)
            gathers[t].wait()

            b, c = t // CPB, t % CPB
            buf = bufs[k]

            def row_body(i, carry, _buf=buf, _p0=c * CH):
                for j in range(D // L):
                    sl = pl.ds(j * L, L)
                    plsc.addupdate(_buf.at[i, sl], pos_v[_p0 + i, sl])
                return carry

            lax.fori_loop(0, CH, row_body, 0)
            out_off = b * S + s0 + c * CH
            stores[t] = pltpu.async_copy(
                buf, out_hbm.at[pl.ds(out_off, CH)], ssems[k])

        for t in range(max(0, NCHT - NBUF), NCHT):
            stores[t].wait()

    return lookup


def kernel(tokens, token_emb, pos_enc):
    B, S = tokens.shape
    D = token_emb.shape[1]
    tok_flat = tokens.reshape(-1).astype(jnp.int32)
    pos2d = pos_enc[0, :S, :]
    out = _make_sc_lookup(B, S, D)(tok_flat, token_emb, pos2d)
    return out.reshape(B, S, D)


# speedup vs baseline: 1.3494x; 1.1163x over previous
import functools

import jax
import jax.numpy as jnp
from jax import lax
from jax.experimental import pallas as pl
from jax.experimental.pallas import tpu as pltpu
from jax.experimental.pallas import tpu_sc as plsc

L = 16
NBUF = 3
CH = 16


@functools.lru_cache(maxsize=None)
def _make_sc_lookup(B, S, D):
    info = plsc.get_sparse_core_info()
    NC, NS = info.num_cores, info.num_subcores
    NW = NC * NS
    N = B * S
    SPW = S // NW
    CPB = SPW // CH
    NCHT = B * CPB
    mesh = plsc.VectorSubcoreMesh(core_axis_name="c", subcore_axis_name="s")

    @functools.partial(
        pl.kernel,
        mesh=mesh,
        out_type=jax.ShapeDtypeStruct((N, D), jnp.float32),
        scratch_types=[
            pltpu.VMEM((B * SPW,), jnp.int32),
            pltpu.VMEM((SPW, D), jnp.float32),
            *[pltpu.VMEM((CH, D), jnp.float32) for _ in range(NBUF)],
            pltpu.SemaphoreType.DMA,
            *[pltpu.SemaphoreType.DMA for _ in range(NBUF)],
            *[pltpu.SemaphoreType.DMA for _ in range(NBUF)],
        ],
    )
    def lookup(tokens_hbm, table_hbm, pos_hbm, out_hbm,
               idx_v, pos_v, *bufs_and_sems):
        bufs = bufs_and_sems[:NBUF]
        pos_sem = bufs_and_sems[NBUF]
        gsems = bufs_and_sems[NBUF + 1:2 * NBUF + 1]
        ssems = bufs_and_sems[2 * NBUF + 1:]

        wid = lax.axis_index("s") * NC + lax.axis_index("c")
        s0 = wid * SPW

        pos_cp = pltpu.async_copy(pos_hbm.at[pl.ds(s0, SPW)], pos_v, pos_sem)
        for b in range(B):
            pltpu.sync_copy(tokens_hbm.at[pl.ds(b * S + s0, SPW)],
                            idx_v.at[pl.ds(b * SPW, SPW)])

        def start_gather(t):
            k = t % NBUF
            return pltpu.async_copy(
                table_hbm.at[idx_v.at[pl.ds(t * CH, CH)]], bufs[k], gsems[k])

        gathers = {t: start_gather(t) for t in range(min(2, NCHT))}
        stores = {}
        pos_cp.wait()

        for t in range(NCHT):
            k = t % NBUF
            gathers[t].wait()

            b, c = t // CPB, t % CPB
            buf = bufs[k]
            p0 = c * CH

            def row_body(i, carry, _buf=buf, _p0=p0):
                for j in range(D // L):
                    sl = pl.ds(j * L, L)
                    plsc.addupdate(_buf.at[i, sl], pos_v[_p0 + i, sl])
                return carry

            lax.fori_loop(0, CH, row_body, 0)
            out_off = b * S + s0 + c * CH
            stores[t] = pltpu.async_copy(
                buf, out_hbm.at[pl.ds(out_off, CH)], ssems[k])
            if t + 2 < NCHT:
                if t + 2 >= NBUF:
                    stores[t + 2 - NBUF].wait()
                gathers[t + 2] = start_gather(t + 2)

        for t in range(max(0, NCHT - NBUF), NCHT):
            stores[t].wait()

    return lookup


def kernel(tokens, token_emb, pos_enc):
    B, S = tokens.shape
    D = token_emb.shape[1]
    tok_flat = tokens.reshape(-1).astype(jnp.int32)
    pos2d = pos_enc[0, :S, :]
    out = _make_sc_lookup(B, S, D)(tok_flat, token_emb, pos2d)
    return out.reshape(B, S, D)


# speedup vs baseline: 1.5807x; 1.1713x over previous
import functools

import jax
import jax.numpy as jnp
from jax import lax
from jax.experimental import pallas as pl
from jax.experimental.pallas import tpu as pltpu
from jax.experimental.pallas import tpu_sc as plsc

L = 16
NBUF = 5
LOOKAHEAD = 3
CH = 16


@functools.lru_cache(maxsize=None)
def _make_sc_lookup(B, S, D):
    info = plsc.get_sparse_core_info()
    NC, NS = info.num_cores, info.num_subcores
    NW = NC * NS
    N = B * S
    SPW = S // NW
    CPB = SPW // CH
    NCHT = B * CPB
    mesh = plsc.VectorSubcoreMesh(core_axis_name="c", subcore_axis_name="s")

    @functools.partial(
        pl.kernel,
        mesh=mesh,
        out_type=jax.ShapeDtypeStruct((N, D), jnp.float32),
        scratch_types=[
            pltpu.VMEM((B * SPW,), jnp.int32),
            *[pltpu.VMEM((CH, D), jnp.float32) for _ in range(2)],
            *[pltpu.VMEM((CH, D), jnp.float32) for _ in range(NBUF)],
            *[pltpu.SemaphoreType.DMA for _ in range(2)],
            *[pltpu.SemaphoreType.DMA for _ in range(NBUF)],
            *[pltpu.SemaphoreType.DMA for _ in range(NBUF)],
        ],
    )
    def lookup(tokens_hbm, table_hbm, pos_hbm, out_hbm,
               idx_v, *refs):
        pps = refs[:2]
        bufs = refs[2:2 + NBUF]
        psems = refs[2 + NBUF:4 + NBUF]
        gsems = refs[4 + NBUF:4 + 2 * NBUF]
        ssems = refs[4 + 2 * NBUF:]

        wid = lax.axis_index("s") * NC + lax.axis_index("c")
        s0 = wid * SPW

        def start_pos(c):
            return pltpu.async_copy(
                pos_hbm.at[pl.ds(s0 + c * CH, CH)], pps[c % 2], psems[c % 2])

        pos_cps = {c: start_pos(c) for c in range(min(2, CPB))}

        for b in range(B):
            pltpu.sync_copy(tokens_hbm.at[pl.ds(b * S + s0, SPW)],
                            idx_v.at[pl.ds(b * SPW, SPW)])

        def start_gather(t):
            c, b = t // B, t % B
            k = t % NBUF
            return pltpu.async_copy(
                table_hbm.at[idx_v.at[pl.ds(b * SPW + c * CH, CH)]],
                bufs[k], gsems[k])

        gathers = {t: start_gather(t) for t in range(min(LOOKAHEAD, NCHT))}
        stores = {}

        for t in range(NCHT):
            c, b = t // B, t % B
            k = t % NBUF
            kp = c % 2
            if b == 0:
                pos_cps[c].wait()
            gathers[t].wait()

            buf = bufs[k]
            pp = pps[kp]

            def row_body(i, carry, _buf=buf, _pp=pp):
                for j in range(D // L):
                    sl = pl.ds(j * L, L)
                    plsc.addupdate(_buf.at[i, sl], _pp[i, sl])
                return carry

            lax.fori_loop(0, CH, row_body, 0)
            out_off = b * S + s0 + c * CH
            stores[t] = pltpu.async_copy(
                buf, out_hbm.at[pl.ds(out_off, CH)], ssems[k])
            if b == B - 1 and c + 2 < CPB:
                pos_cps[c + 2] = start_pos(c + 2)
            if t + LOOKAHEAD < NCHT:
                if t + LOOKAHEAD >= NBUF:
                    stores[t + LOOKAHEAD - NBUF].wait()
                gathers[t + LOOKAHEAD] = start_gather(t + LOOKAHEAD)

        for t in range(max(0, NCHT - NBUF), NCHT):
            stores[t].wait()

    return lookup


def kernel(tokens, token_emb, pos_enc):
    B, S = tokens.shape
    D = token_emb.shape[1]
    tok_flat = tokens.reshape(-1).astype(jnp.int32)
    pos2d = pos_enc[0, :S, :]
    out = _make_sc_lookup(B, S, D)(tok_flat, token_emb, pos2d)
    return out.reshape(B, S, D)
